# baseline (speedup 1.0000x reference)
import functools

import jax
import jax.numpy as jnp
from jax import lax
from jax.experimental import pallas as pl
from jax.experimental.pallas import tpu as pltpu
from jax.experimental.pallas import tpu_sc as plsc

N = 10000
E = 320000
D = 128
R = 64
RANK = 8

NC = 2
NS = 16
NW = NC * NS
CHUNK = 128
NACC = N + 112
BLK = NACC // 8
GRID = NACC // BLK

F32 = jnp.float32


def _ceil_to(x, m):
    return (x + m - 1) // m * m


def _make_segsum(rows_pw, do_gather=True):
    mesh = plsc.VectorSubcoreMesh(core_axis_name="c", subcore_axis_name="s")
    rpt = NACC // NS

    IH = 2 if rows_pw >= 16 else 1
    hr = rows_pw // IH
    NB = 2
    assert hr % NB == 0 and (IH == 1 or hr % 8 == 0)

    @functools.partial(
        pl.kernel,
        out_type=jax.ShapeDtypeStruct((NC, NACC, D), F32),
        mesh=mesh,
        scratch_types=(
            [pltpu.VMEM((hr, CHUNK), jnp.int32),
             pltpu.VMEM((hr, CHUNK), jnp.int32)]
            + [pltpu.VMEM((CHUNK, D), F32)] * NB
            + [pltpu.VMEM_SHARED((NACC, D), F32)]
            + [pltpu.SemaphoreType.DMA] * NB
        ),
    )
    def k(table_h, src_h, dst_h, zero_h, out_h, src_v, dst_v, *rest):
        rows = rest[:NB]
        acc_sh = rest[NB]
        sems = rest[NB + 1:]
        c = lax.axis_index("c")
        s = lax.axis_index("s")
        wid = s * NC + c
        pltpu.sync_copy(zero_h.at[pl.ds(s * rpt, rpt)],
                        acc_sh.at[pl.ds(s * rpt, rpt)])
        if not do_gather:
            pltpu.sync_copy(table_h.at[pl.ds(0, CHUNK)], rows[0])
        plsc.subcore_barrier()

        for h in range(IH):
            base = wid * rows_pw + h * hr
            pltpu.sync_copy(src_h.at[pl.ds(base, hr)], src_v)
            pltpu.sync_copy(dst_h.at[pl.ds(base, hr)], dst_v)
            if do_gather:
                for b in range(NB):
                    pltpu.async_copy(table_h.at[src_v.at[b]], rows[b], sems[b])

                def group(g, carry):
                    for b in range(NB):
                        j = g * NB + b
                        pltpu.make_async_copy(
                            table_h.at[src_v.at[j]], rows[b], sems[b]).wait()
                        pltpu.sync_copy(rows[b], acc_sh.at[dst_v.at[j]],
                                        add=True)
                        jn = jnp.minimum(j + NB, hr - 1)
                        pltpu.async_copy(table_h.at[src_v.at[jn]], rows[b],
                                         sems[b])
                    return carry

                lax.fori_loop(0, hr // NB, group, 0)
                for b in range(NB):
                    pltpu.make_async_copy(
                        table_h.at[src_v.at[0]], rows[b], sems[b]).wait()
            else:
                def group(g, carry):
                    for b in range(NB):
                        pltpu.async_copy(
                            rows[0], acc_sh.at[dst_v.at[g * NB + b]], sems[b],
                            add=True)
                    for b in range(NB):
                        pltpu.make_async_copy(
                            rows[0], acc_sh.at[dst_v.at[g * NB + b]],
                            sems[b]).wait()
                    return carry

                lax.fori_loop(0, hr // NB, group, 0)

        plsc.subcore_barrier()
        pltpu.sync_copy(acc_sh.at[pl.ds(s * rpt, rpt)],
                        out_h.at[c, pl.ds(s * rpt, rpt)])

    return k


def _pad_edges(src, dst, rows_pw):
    ep = NW * rows_pw * CHUNK
    e = src.shape[0]
    srcp = jnp.concatenate([src, jnp.zeros((ep - e,), jnp.int32)])
    dstp = jnp.concatenate([dst, jnp.full((ep - e,), N, jnp.int32)])
    return srcp.reshape(-1, CHUNK), dstp.reshape(-1, CHUNK)


def _ln(x, eps=1e-5):
    m = jnp.mean(x, axis=-1, keepdims=True)
    v = jnp.var(x, axis=-1, keepdims=True)
    return (x - m) / jnp.sqrt(v + eps)


def _dot(a, b):
    return jnp.dot(a, b, preferred_element_type=F32)


def _row_spec(w):
    return pl.BlockSpec((1, BLK, w), lambda i: (0, i, 0))


def _rows2_spec(w):
    return pl.BlockSpec((2, BLK, w), lambda i: (0, i, 0))


def _rep_spec(shape):
    nd = len(shape)
    return pl.BlockSpec(shape, lambda i, _n=nd: (0,) * _n)


def _tc_rows(fn, out_widths, bigs, smalls, name):
    in_specs, args = [], []
    for a in bigs:
        if a.ndim == 3:
            in_specs.append(_rows2_spec(a.shape[2]))
            args.append(a)
        else:
            in_specs.append(_row_spec(a.shape[1]))
            args.append(a[None])
    for sm in smalls:
        in_specs.append(_rep_spec(sm.shape))
        args.append(sm)
    out_shape = [jax.ShapeDtypeStruct((NACC, w), F32) for w in out_widths]
    out_specs = [pl.BlockSpec((BLK, w), lambda i: (i, 0)) for w in out_widths]
    return pl.pallas_call(
        fn, grid=(GRID,), in_specs=in_specs, out_specs=out_specs,
        out_shape=out_shape, name=name)(*args)


def _stage_init(pe, degp, Wg1):
    def f(pe_r, dp_r, W_r, o_init, o_g, o_c, o_hs):
        init = pe_r[0] + pe_r[1]
        deg = dp_r[0, :, :1] + dp_r[1, :, :1]
        disg = lax.rsqrt(deg + 1.0)
        disc = jnp.where(deg > 0, lax.rsqrt(jnp.maximum(deg, 1e-12)), 0.0)
        o_init[...] = init
        o_g[...] = disg
        o_c[...] = disc
        o_hs[...] = disg * _dot(init, W_r[...])
    return _tc_rows(f, [D, 1, 1, D], [pe, degp], [Wg1], "stage_init")


def _stage_gcn(p, hs, disg, b2d, Wn, bn2d, with_k):
    if with_k:
        def f(p_r, hs_r, g_r, b_r, W_r, bn_r, o_x, o_k):
            x = jax.nn.gelu(_ln(g_r[0] * (p_r[0] + p_r[1] + hs_r[0])
                                + b_r[...]))
            o_x[...] = x
            o_k[...] = _dot(x, W_r[...]) + bn_r[...]
        return _tc_rows(f, [D, D], [p, hs, disg], [b2d, Wn, bn2d],
                        "stage_gcn2")

    def f(p_r, hs_r, g_r, b_r, W_r, o_hs):
        x = jax.nn.gelu(_ln(g_r[0] * (p_r[0] + p_r[1] + hs_r[0])
                            + b_r[...]))
        o_hs[...] = g_r[0] * _dot(x, W_r[...])
    return _tc_rows(f, [D], [p, hs, disg], [b2d, Wn], "stage_gcn1")


def _attn_pool(kk, seg_h, cc, Wq, bq2d, Wr1, br12d, Wr2, br22d):
    scale = 1.0 / (D ** 0.5)

    def f(kk_r, v_r, cc_r, Wq_r, bq_r, W1_r, b1_r, W2_r, b2_r,
          proj_o, m_o, z_o, rf_s, m_s, z_s):
        i = pl.program_id(0)

        @pl.when(i == 0)
        def _():
            m_s[...] = jnp.full((R, 1), -1e30, F32)
            z_s[...] = jnp.zeros((R, 1), F32)
            rf_s[...] = jnp.zeros((R, D), F32)

        q = _dot(cc_r[...], Wq_r[...]) + bq_r[...]
        s = lax.dot_general(q, kk_r[0], (((1,), (1,)), ((), ())),
                            preferred_element_type=F32) * scale
        col = lax.broadcasted_iota(jnp.int32, (1, BLK), 1) + i * BLK
        s = jnp.where(col < N, s, -1e30)
        m_old = m_s[...]
        m_new = jnp.maximum(m_old, jnp.max(s, axis=1, keepdims=True))
        alpha = jnp.exp(m_old - m_new)
        p = jnp.exp(s - m_new)
        z_s[...] = z_s[...] * alpha + jnp.sum(p, axis=1, keepdims=True)
        rf_s[...] = rf_s[...] * alpha + _dot(p, v_r[0])
        m_s[...] = m_new

        @pl.when(i == GRID - 1)
        def _():
            rf = rf_s[...] / z_s[...] + cc_r[...]
            pr = _dot(jax.nn.gelu(_dot(rf, W1_r[...]) + b1_r[...]),
                      W2_r[...]) + b2_r[...]
            proj_o[...] = _ln(pr)
            m_o[...] = m_s[...]
            z_o[...] = z_s[...]

    out_shape = [jax.ShapeDtypeStruct((R, D), F32),
                 jax.ShapeDtypeStruct((R, 1), F32),
                 jax.ShapeDtypeStruct((R, 1), F32)]
    return pl.pallas_call(
        f, grid=(GRID,),
        in_specs=[_row_spec(D), _row_spec(D), _rep_spec((R, D)),
                  _rep_spec((D, D)), _rep_spec((1, D)), _rep_spec((D, D)),
                  _rep_spec((1, D)), _rep_spec((D, D)), _rep_spec((1, D))],
        out_specs=[_rep_spec((R, D)), _rep_spec((R, 1)), _rep_spec((R, 1))],
        out_shape=out_shape,
        scratch_shapes=[pltpu.VMEM((R, D), F32), pltpu.VMEM((R, 1), F32),
                        pltpu.VMEM((R, 1), F32)],
        name="attn_pool")(kk[None], seg_h[None], cc, Wq, bq2d, Wr1, br12d,
                          Wr2, br22d)


def _attn_unpool(kk, init, disc, proj, m, z, cc, Wq, bq2d,
                 W0l, bl2d, W0h, bh2d, gam2d, bet2d):
    scale = 1.0 / (D ** 0.5)

    def f(kk_r, init_r, c_r, proj_r, m_r, z_r, cc_r, Wq_r, bq_r,
          W0l_r, bl_r, W0h_r, bh_r, gam_r, bet_r,
          o_slr, o_slrs, o_resid, o_resids, o_accl, o_acch):
        q = _dot(cc_r[...], Wq_r[...]) + bq_r[...]
        s = lax.dot_general(q, kk_r[0], (((1,), (1,)), ((), ())),
                            preferred_element_type=F32) * scale
        a = jnp.exp(s - m_r[...]) / z_r[...]
        slr = lax.dot_general(a, proj_r[...], (((0,), (0,)), ((), ())),
                              preferred_element_type=F32)
        resid = (init_r[0] - slr) * gam_r[...] + bet_r[...]
        o_slr[...] = slr
        o_slrs[...] = c_r[0] * slr
        o_resid[...] = resid
        o_resids[...] = c_r[0] * resid
        o_accl[...] = _dot(slr, W0l_r[...]) + bl_r[...]
        o_acch[...] = _dot(resid, W0h_r[...]) + bh_r[...]

    return _tc_rows(f, [D, D, D, D, D, D], [kk, init, disc],
                    [proj, m, z, cc, Wq, bq2d, W0l, bl2d, W0h, bh2d,
                     gam2d, bet2d], "attn_unpool")


def _cheb_step(p, disc, acc, sub, Wk, coef):
    if sub is None:
        def f(p_r, c_r, acc_r, W_r, o_tx, o_txs, o_acc):
            tx = (-coef) * c_r[0] * (p_r[0] + p_r[1])
            o_tx[...] = tx
            o_txs[...] = c_r[0] * tx
            o_acc[...] = acc_r[0] + _dot(tx, W_r[...])
        return _tc_rows(f, [D, D, D], [p, disc, acc], [Wk], "cheb_step0")

    def f(p_r, c_r, acc_r, sub_r, W_r, o_tx, o_txs, o_acc):
        tx = (-coef) * c_r[0] * (p_r[0] + p_r[1]) - sub_r[0]
        o_tx[...] = tx
        o_txs[...] = c_r[0] * tx
        o_acc[...] = acc_r[0] + _dot(tx, W_r[...])
    return _tc_rows(f, [D, D, D], [p, disc, acc, sub], [Wk], "cheb_step")


def _cheb_last(p, disc, acc, sub, Wk, coef):
    def f(p_r, c_r, acc_r, sub_r, W_r, o_out):
        tx = (-coef) * c_r[0] * (p_r[0] + p_r[1]) - sub_r[0]
        o_out[...] = _ln(jax.nn.gelu(acc_r[0] + _dot(tx, W_r[...])))
    return _tc_rows(f, [D], [p, disc, acc, sub], [Wk], "cheb_last")[0]


def _stage_final(seg_low, high, WgaA, WgaB, bga2d, Wgb, bgb2d,
                 Wo1, bo12d, Wo2, bo22d):
    def f(sl_r, hi_r, WA_r, WB_r, ba_r, Wg_r, bg_r, W1_r, b1_r, W2_r, b2_r,
          o_out):
        sl = sl_r[0]
        hi = hi_r[0]
        g1 = jax.nn.gelu(_dot(sl, WA_r[...]) + _dot(hi, WB_r[...]) + ba_r[...])
        gate = jax.nn.sigmoid(_dot(g1, Wg_r[...]) + bg_r[...])
        fused = gate * sl + (1.0 - gate) * hi
        o_out[...] = _dot(jax.nn.gelu(_dot(fused, W1_r[...]) + b1_r[...]),
                          W2_r[...]) + b2_r[...]
    return _tc_rows(f, [D], [seg_low, high],
                    [WgaA, WgaB, bga2d, Wgb, bgb2d, Wo1, bo12d, Wo2, bo22d],
                    "stage_final")[0]


def kernel(segment_features, edge_index, city_idx, lane_emb, type_emb,
           length_emb, Wp, bp, Wg1, bg1, Wg2, bg2, centers, city_emb,
           adapter_W, Wq, bq, Wk, bk, Wr1, br1, Wr2, br2, Wc_low, bc_low,
           Wfilm, Wc_high, bc_high, Wga, bga, Wgb, bgb, Wo1, bo1, Wo2, bo2):
    src = edge_index[0].astype(jnp.int32)
    dst = edge_index[1].astype(jnp.int32)
    sf = segment_features.astype(jnp.int32)

    ce = city_emb[city_idx]
    cc = centers + (ce @ adapter_W).reshape(R, D)
    gb = ce @ Wfilm
    gam2d = (1.0 + gb[:D])[None, :]
    bet2d = gb[D:][None, :]
    r2 = lambda v: v.reshape(1, -1)
    t_emb = jnp.concatenate([
        lane_emb @ Wp[:32] + bp,
        type_emb @ Wp[32:64],
        length_emb @ Wp[64:],
    ], axis=0)

    rows_main = _ceil_to(_ceil_to(E, NW * CHUNK) // (NW * CHUNK), 8)
    srcR, dstR = _pad_edges(src, dst, rows_main)
    zeros128 = jnp.zeros((NACC, D), F32)
    segsum_main = _make_segsum(rows_main)

    def segsum(table):
        return segsum_main(table, srcR, dstR, zeros128)

    ones128 = jnp.ones((CHUNK, D), F32)
    degp = _make_segsum(rows_main, do_gather=False)(
        ones128, srcR, dstR, zeros128)

    idx_e = jnp.concatenate([sf[:, 0], sf[:, 1] + 16, sf[:, 2] + 48])
    ar = jnp.arange(N, dtype=jnp.int32)
    nodes = jnp.concatenate([ar, ar, ar])
    rows_emb = _ceil_to(3 * N, NW * CHUNK) // (NW * CHUNK)
    srcE, dstE = _pad_edges(idx_e, nodes, rows_emb)
    pe = _make_segsum(rows_emb)(t_emb, srcE, dstE, zeros128)

    init, disg, disc, hs1 = _stage_init(pe, degp, Wg1)

    p1 = segsum(hs1)
    hs2 = _stage_gcn(p1, hs1, disg, r2(bg1), Wg2, None, False)[0]
    p2 = segsum(hs2)
    seg_h, kk = _stage_gcn(p2, hs2, disg, r2(bg2), Wk, r2(bk), True)

    proj, m, z = _attn_pool(kk, seg_h, cc, Wq, r2(bq), Wr1, r2(br1),
                            Wr2, r2(br2))
    slr, slrs, resid, resids, accl, acch = _attn_unpool(
        kk, init, disc, proj, m, z, cc, Wq, r2(bq),
        Wc_low[0], r2(bc_low), Wc_high[0], r2(bc_high), gam2d, bet2d)

    tx1, txs1, accl = _cheb_step(segsum(slrs), disc, accl, None,
                                 Wc_low[1], 1.0)
    tx2, txs2, accl = _cheb_step(segsum(txs1), disc, accl, slr,
                                 Wc_low[2], 2.0)
    tx3, txs3, accl = _cheb_step(segsum(txs2), disc, accl, tx1,
                                 Wc_low[3], 2.0)
    seg_low = _cheb_last(segsum(txs3), disc, accl, tx2, Wc_low[4], 2.0)

    th1, ths1, acch = _cheb_step(segsum(resids), disc, acch, None,
                                 Wc_high[1], 1.0)
    high = _cheb_last(segsum(ths1), disc, acch, resid, Wc_high[2], 2.0)

    out = _stage_final(seg_low, high, Wga[:D], Wga[D:], r2(bga), Wgb,
                       r2(bgb), Wo1, r2(bo1), Wo2, r2(bo2))
    return out[:N]


# speedup vs baseline: 1.0545x; 1.0545x over previous
import functools

import jax
import jax.numpy as jnp
from jax import lax
from jax.experimental import pallas as pl
from jax.experimental.pallas import tpu as pltpu
from jax.experimental.pallas import tpu_sc as plsc

N = 10000
E = 320000
D = 128
R = 64
RANK = 8

NC = 2
NS = 16
NW = NC * NS
CHUNK = 128
NACC = N + 112
BLK = NACC // 8
GRID = NACC // BLK

F32 = jnp.float32


def _ceil_to(x, m):
    return (x + m - 1) // m * m


def _make_segsum(rows_pw, do_gather=True):
    mesh = plsc.VectorSubcoreMesh(core_axis_name="c", subcore_axis_name="s")
    rpt = NACC // NS

    IH = 2 if rows_pw >= 16 else 1
    hr = rows_pw // IH
    NB = 2
    assert hr % NB == 0 and (IH == 1 or hr % 8 == 0)

    @functools.partial(
        pl.kernel,
        out_type=jax.ShapeDtypeStruct((NC, NACC, D), F32),
        mesh=mesh,
        scratch_types=(
            [pltpu.VMEM((hr, CHUNK), jnp.int32),
             pltpu.VMEM((hr, CHUNK), jnp.int32)]
            + [pltpu.VMEM((CHUNK, D), F32)] * NB
            + [pltpu.VMEM_SHARED((NACC, D), F32)]
            + [pltpu.SemaphoreType.DMA] * NB
        ),
    )
    def k(table_h, src_h, dst_h, zero_h, out_h, src_v, dst_v, *rest):
        rows = rest[:NB]
        acc_sh = rest[NB]
        sems = rest[NB + 1:]
        c = lax.axis_index("c")
        s = lax.axis_index("s")
        wid = s * NC + c
        pltpu.sync_copy(zero_h.at[pl.ds(s * rpt, rpt)],
                        acc_sh.at[pl.ds(s * rpt, rpt)])
        if not do_gather:
            pltpu.sync_copy(table_h.at[pl.ds(0, CHUNK)], rows[0])
        plsc.subcore_barrier()

        for h in range(IH):
            base = wid * rows_pw + h * hr
            pltpu.sync_copy(src_h.at[pl.ds(base, hr)], src_v)
            pltpu.sync_copy(dst_h.at[pl.ds(base, hr)], dst_v)
            if do_gather:
                for b in range(NB):
                    pltpu.async_copy(table_h.at[src_v.at[b]], rows[b], sems[b])

                def group(g, carry):
                    for b in range(NB):
                        j = g * NB + b
                        pltpu.make_async_copy(
                            table_h.at[src_v.at[j]], rows[b], sems[b]).wait()
                        pltpu.sync_copy(rows[b], acc_sh.at[dst_v.at[j]],
                                        add=True)
                        jn = jnp.minimum(j + NB, hr - 1)
                        pltpu.async_copy(table_h.at[src_v.at[jn]], rows[b],
                                         sems[b])
                    return carry

                lax.fori_loop(0, hr // NB, group, 0)
                for b in range(NB):
                    pltpu.make_async_copy(
                        table_h.at[src_v.at[0]], rows[b], sems[b]).wait()
            else:
                def group(g, carry):
                    for b in range(NB):
                        pltpu.async_copy(
                            rows[0], acc_sh.at[dst_v.at[g * NB + b]], sems[b],
                            add=True)
                    for b in range(NB):
                        pltpu.make_async_copy(
                            rows[0], acc_sh.at[dst_v.at[g * NB + b]],
                            sems[b]).wait()
                    return carry

                lax.fori_loop(0, hr // NB, group, 0)

        plsc.subcore_barrier()
        pltpu.sync_copy(acc_sh.at[pl.ds(s * rpt, rpt)],
                        out_h.at[c, pl.ds(s * rpt, rpt)])

    return k


def _pad_edges(src, dst, rows_pw):
    ep = NW * rows_pw * CHUNK
    e = src.shape[0]
    srcp = jnp.concatenate([src, jnp.zeros((ep - e,), jnp.int32)])
    dstp = jnp.concatenate([dst, jnp.full((ep - e,), N, jnp.int32)])
    return srcp.reshape(-1, CHUNK), dstp.reshape(-1, CHUNK)


def _ln(x, eps=1e-5):
    m = jnp.mean(x, axis=-1, keepdims=True)
    v = jnp.var(x, axis=-1, keepdims=True)
    return (x - m) / jnp.sqrt(v + eps)


def _dot(a, b):
    return jnp.dot(a, b, preferred_element_type=F32)


def _row_spec(w):
    return pl.BlockSpec((1, BLK, w), lambda i: (0, i, 0))


def _rows2_spec(w):
    return pl.BlockSpec((2, BLK, w), lambda i: (0, i, 0))


def _rep_spec(shape):
    nd = len(shape)
    return pl.BlockSpec(shape, lambda i, _n=nd: (0,) * _n)


def _tc_rows(fn, out_widths, bigs, smalls, name):
    in_specs, args = [], []
    for a in bigs:
        if a.ndim == 3:
            in_specs.append(_rows2_spec(a.shape[2]))
            args.append(a)
        else:
            in_specs.append(_row_spec(a.shape[1]))
            args.append(a[None])
    for sm in smalls:
        in_specs.append(_rep_spec(sm.shape))
        args.append(sm)
    out_shape = [jax.ShapeDtypeStruct((NACC, w), F32) for w in out_widths]
    out_specs = [pl.BlockSpec((BLK, w), lambda i: (i, 0)) for w in out_widths]
    return pl.pallas_call(
        fn, grid=(GRID,), in_specs=in_specs, out_specs=out_specs,
        out_shape=out_shape, name=name)(*args)


def _stage_init(pe, degp, Wg1):
    def f(pe_r, dp_r, W_r, o_init, o_g, o_c, o_hs):
        init = pe_r[0] + pe_r[1]
        deg = dp_r[0, :, :1] + dp_r[1, :, :1]
        disg = lax.rsqrt(deg + 1.0)
        disc = jnp.where(deg > 0, lax.rsqrt(jnp.maximum(deg, 1e-12)), 0.0)
        o_init[...] = init
        o_g[...] = disg
        o_c[...] = disc
        o_hs[...] = disg * _dot(init, W_r[...])
    return _tc_rows(f, [D, 1, 1, D], [pe, degp], [Wg1], "stage_init")


def _stage_gcn(p, hs, disg, b2d, Wn, bn2d, with_k):
    if with_k:
        def f(p_r, hs_r, g_r, b_r, W_r, bn_r, o_x, o_k):
            x = jax.nn.gelu(_ln(g_r[0] * (p_r[0] + p_r[1] + hs_r[0])
                                + b_r[...]))
            o_x[...] = x
            o_k[...] = _dot(x, W_r[...]) + bn_r[...]
        return _tc_rows(f, [D, D], [p, hs, disg], [b2d, Wn, bn2d],
                        "stage_gcn2")

    def f(p_r, hs_r, g_r, b_r, W_r, o_hs):
        x = jax.nn.gelu(_ln(g_r[0] * (p_r[0] + p_r[1] + hs_r[0])
                            + b_r[...]))
        o_hs[...] = g_r[0] * _dot(x, W_r[...])
    return _tc_rows(f, [D], [p, hs, disg], [b2d, Wn], "stage_gcn1")


def _attn_pool(kk, seg_h, cc, Wq, bq2d, Wr1, br12d, Wr2, br22d):
    scale = 1.0 / (D ** 0.5)

    def f(kk_r, v_r, cc_r, Wq_r, bq_r, W1_r, b1_r, W2_r, b2_r,
          proj_o, m_o, z_o, rf_s, m_s, z_s):
        i = pl.program_id(0)

        @pl.when(i == 0)
        def _():
            m_s[...] = jnp.full((R, 1), -1e30, F32)
            z_s[...] = jnp.zeros((R, 1), F32)
            rf_s[...] = jnp.zeros((R, D), F32)

        q = _dot(cc_r[...], Wq_r[...]) + bq_r[...]
        s = lax.dot_general(q, kk_r[0], (((1,), (1,)), ((), ())),
                            preferred_element_type=F32) * scale
        col = lax.broadcasted_iota(jnp.int32, (1, BLK), 1) + i * BLK
        s = jnp.where(col < N, s, -1e30)
        m_old = m_s[...]
        m_new = jnp.maximum(m_old, jnp.max(s, axis=1, keepdims=True))
        alpha = jnp.exp(m_old - m_new)
        p = jnp.exp(s - m_new)
        z_s[...] = z_s[...] * alpha + jnp.sum(p, axis=1, keepdims=True)
        rf_s[...] = rf_s[...] * alpha + _dot(p, v_r[0])
        m_s[...] = m_new

        @pl.when(i == GRID - 1)
        def _():
            rf = rf_s[...] / z_s[...] + cc_r[...]
            pr = _dot(jax.nn.gelu(_dot(rf, W1_r[...]) + b1_r[...]),
                      W2_r[...]) + b2_r[...]
            proj_o[...] = _ln(pr)
            m_o[...] = m_s[...]
            z_o[...] = z_s[...]

    out_shape = [jax.ShapeDtypeStruct((R, D), F32),
                 jax.ShapeDtypeStruct((R, 1), F32),
                 jax.ShapeDtypeStruct((R, 1), F32)]
    return pl.pallas_call(
        f, grid=(GRID,),
        in_specs=[_row_spec(D), _row_spec(D), _rep_spec((R, D)),
                  _rep_spec((D, D)), _rep_spec((1, D)), _rep_spec((D, D)),
                  _rep_spec((1, D)), _rep_spec((D, D)), _rep_spec((1, D))],
        out_specs=[_rep_spec((R, D)), _rep_spec((R, 1)), _rep_spec((R, 1))],
        out_shape=out_shape,
        scratch_shapes=[pltpu.VMEM((R, D), F32), pltpu.VMEM((R, 1), F32),
                        pltpu.VMEM((R, 1), F32)],
        name="attn_pool")(kk[None], seg_h[None], cc, Wq, bq2d, Wr1, br12d,
                          Wr2, br22d)


def _attn_unpool(kk, init, disc, proj, m, z, cc, Wq, bq2d,
                 gam2d, bet2d):
    scale = 1.0 / (D ** 0.5)

    def f(kk_r, init_r, c_r, proj_r, m_r, z_r, cc_r, Wq_r, bq_r,
          gam_r, bet_r, o_slr, o_slrs, o_resid, o_resids):
        q = _dot(cc_r[...], Wq_r[...]) + bq_r[...]
        s = lax.dot_general(q, kk_r[0], (((1,), (1,)), ((), ())),
                            preferred_element_type=F32) * scale
        a = jnp.exp(s - m_r[...]) / z_r[...]
        slr = lax.dot_general(a, proj_r[...], (((0,), (0,)), ((), ())),
                              preferred_element_type=F32)
        resid = (init_r[0] - slr) * gam_r[...] + bet_r[...]
        o_slr[...] = slr
        o_slrs[...] = c_r[0] * slr
        o_resid[...] = resid
        o_resids[...] = c_r[0] * resid

    return _tc_rows(f, [D, D, D, D], [kk, init, disc],
                    [proj, m, z, cc, Wq, bq2d, gam2d, bet2d], "attn_unpool")


def _cheb_step(p, disc, sub, coef):
    if sub is None:
        def f(p_r, c_r, o_tx, o_txs):
            tx = (-coef) * c_r[0] * (p_r[0] + p_r[1])
            o_tx[...] = tx
            o_txs[...] = c_r[0] * tx
        return _tc_rows(f, [D, D], [p, disc], [], "cheb_step0")

    def f(p_r, c_r, sub_r, o_tx, o_txs):
        tx = (-coef) * c_r[0] * (p_r[0] + p_r[1]) - sub_r[0]
        o_tx[...] = tx
        o_txs[...] = c_r[0] * tx
    return _tc_rows(f, [D, D], [p, disc, sub], [], "cheb_step")


def _cheb_final(p, disc, sub, txs, Ws, b2d):
    K = len(txs) + 1

    def f(*refs):
        p_r, c_r, sub_r = refs[0], refs[1], refs[2]
        tx_rs = refs[3:3 + len(txs)]
        W_r, b_r = refs[3 + len(txs)], refs[4 + len(txs)]
        o_out = refs[5 + len(txs)]
        txk = -2.0 * c_r[0] * (p_r[0] + p_r[1]) - sub_r[0]
        acc = b_r[...] + _dot(txk, W_r[K - 1])
        for k in range(K - 1):
            acc = acc + _dot(tx_rs[k][0], W_r[k])
        o_out[...] = _ln(jax.nn.gelu(acc))

    return _tc_rows(f, [D], [p, disc, sub] + list(txs), [Ws, b2d],
                    "cheb_final")[0]


def _stage_final(seg_low, high, WgaA, WgaB, bga2d, Wgb, bgb2d,
                 Wo1, bo12d, Wo2, bo22d):
    def f(sl_r, hi_r, WA_r, WB_r, ba_r, Wg_r, bg_r, W1_r, b1_r, W2_r, b2_r,
          o_out):
        sl = sl_r[0]
        hi = hi_r[0]
        g1 = jax.nn.gelu(_dot(sl, WA_r[...]) + _dot(hi, WB_r[...]) + ba_r[...])
        gate = jax.nn.sigmoid(_dot(g1, Wg_r[...]) + bg_r[...])
        fused = gate * sl + (1.0 - gate) * hi
        o_out[...] = _dot(jax.nn.gelu(_dot(fused, W1_r[...]) + b1_r[...]),
                          W2_r[...]) + b2_r[...]
    return _tc_rows(f, [D], [seg_low, high],
                    [WgaA, WgaB, bga2d, Wgb, bgb2d, Wo1, bo12d, Wo2, bo22d],
                    "stage_final")[0]


def kernel(segment_features, edge_index, city_idx, lane_emb, type_emb,
           length_emb, Wp, bp, Wg1, bg1, Wg2, bg2, centers, city_emb,
           adapter_W, Wq, bq, Wk, bk, Wr1, br1, Wr2, br2, Wc_low, bc_low,
           Wfilm, Wc_high, bc_high, Wga, bga, Wgb, bgb, Wo1, bo1, Wo2, bo2):
    src = edge_index[0].astype(jnp.int32)
    dst = edge_index[1].astype(jnp.int32)
    sf = segment_features.astype(jnp.int32)

    ce = city_emb[city_idx]
    cc = centers + (ce @ adapter_W).reshape(R, D)
    gb = ce @ Wfilm
    gam2d = (1.0 + gb[:D])[None, :]
    bet2d = gb[D:][None, :]
    r2 = lambda v: v.reshape(1, -1)
    t_emb = jnp.concatenate([
        lane_emb @ Wp[:32] + bp,
        type_emb @ Wp[32:64],
        length_emb @ Wp[64:],
    ], axis=0)

    rows_main = _ceil_to(_ceil_to(E, NW * CHUNK) // (NW * CHUNK), 8)
    srcR, dstR = _pad_edges(src, dst, rows_main)
    zeros128 = jnp.zeros((NACC, D), F32)
    segsum_main = _make_segsum(rows_main)

    def segsum(table):
        return segsum_main(table, srcR, dstR, zeros128)

    ones128 = jnp.ones((CHUNK, D), F32)
    degp = _make_segsum(rows_main, do_gather=False)(
        ones128, srcR, dstR, zeros128)

    idx_e = jnp.concatenate([sf[:, 0], sf[:, 1] + 16, sf[:, 2] + 48])
    ar = jnp.arange(N, dtype=jnp.int32)
    nodes = jnp.concatenate([ar, ar, ar])
    rows_emb = _ceil_to(3 * N, NW * CHUNK) // (NW * CHUNK)
    srcE, dstE = _pad_edges(idx_e, nodes, rows_emb)
    pe = _make_segsum(rows_emb)(t_emb, srcE, dstE, zeros128)

    init, disg, disc, hs1 = _stage_init(pe, degp, Wg1)

    p1 = segsum(hs1)
    hs2 = _stage_gcn(p1, hs1, disg, r2(bg1), Wg2, None, False)[0]
    p2 = segsum(hs2)
    seg_h, kk = _stage_gcn(p2, hs2, disg, r2(bg2), Wk, r2(bk), True)

    proj, m, z = _attn_pool(kk, seg_h, cc, Wq, r2(bq), Wr1, r2(br1),
                            Wr2, r2(br2))
    slr, slrs, resid, resids = _attn_unpool(
        kk, init, disc, proj, m, z, cc, Wq, r2(bq), gam2d, bet2d)

    tx1, txs1 = _cheb_step(segsum(slrs), disc, None, 1.0)
    tx2, txs2 = _cheb_step(segsum(txs1), disc, slr, 2.0)
    tx3, txs3 = _cheb_step(segsum(txs2), disc, tx1, 2.0)
    seg_low = _cheb_final(segsum(txs3), disc, tx2, [slr, tx1, tx2, tx3],
                          Wc_low, r2(bc_low))

    th1, ths1 = _cheb_step(segsum(resids), disc, None, 1.0)
    high = _cheb_final(segsum(ths1), disc, resid, [resid, th1],
                       Wc_high, r2(bc_high))

    out = _stage_final(seg_low, high, Wga[:D], Wga[D:], r2(bga), Wgb,
                       r2(bgb), Wo1, r2(bo1), Wo2, r2(bo2))
    return out[:N]
